# Initial kernel scaffold; baseline (speedup 1.0000x reference)
import jax
import jax.numpy as jnp
from jax.experimental import pallas as pl


def kernel(x, edge_index, batch, W1, b1, g1, bt1, W2, b2, g2, bt2, W3, b3, g3, bt3, Wc, bc):
    raise NotImplementedError("write your pallas kernel here")


# speedup vs baseline: 8.2294x; 8.2294x over previous
import functools

import jax
import jax.numpy as jnp
from jax import lax
from jax.experimental import pallas as pl
from jax.experimental.pallas import tpu as pltpu
from jax.experimental.pallas import tpu_sc as plsc

N_NODES = 10000
N_EDGES = 320000
D_IN = 128
D_H = 256
N_GRAPHS = 64
EPS = 1e-5

NROWS = 10240
ROWS_PER_TILE = NROWS // 16
TRASH = 10008
AGG_K = 128
AGG_CHUNKS = 157
E_PAD = 16 * AGG_CHUNKS * AGG_K
DEG_K = 64
DEG_CHUNKS = 157
DEG_PER_TILE = E_PAD // 32

_HIGH = lax.Precision.HIGHEST


def _dot(a, b):
    return jnp.dot(a, b, precision=_HIGH, preferred_element_type=jnp.float32)


BLK = 1000
GRID = N_NODES // BLK


def _dinv_block(degp_b):
    deg = degp_b[0, :, 0:1] + degp_b[1, :, 0:1] + 1.0
    return lax.rsqrt(deg)


def _tc_a_body(x_ref, w1_ref, degp_ref, hs_ref):
    dinv = _dinv_block(degp_ref[...])
    hs = _dot(x_ref[...], w1_ref[...]) * dinv
    hs_ref[0] = hs[:, :128]
    hs_ref[1] = hs[:, 128:]


def _q_block(agg_ref, hs_ref, dinv, p):
    return dinv * (agg_ref[p] + hs_ref[p])


def _stats_body(agg_ref, hs_ref, degp_ref, sum_ref, sq_ref):
    i = pl.program_id(0)
    dinv = _dinv_block(degp_ref[...])
    q0 = _q_block(agg_ref, hs_ref, dinv, 0)
    q1 = _q_block(agg_ref, hs_ref, dinv, 1)
    s = jnp.concatenate([jnp.sum(q0, axis=0, keepdims=True),
                         jnp.sum(q1, axis=0, keepdims=True)], axis=0)
    sq = jnp.concatenate([jnp.sum(q0 * q0, axis=0, keepdims=True),
                          jnp.sum(q1 * q1, axis=0, keepdims=True)], axis=0)

    @pl.when(i == 0)
    def _():
        sum_ref[...] = s
        sq_ref[...] = sq

    @pl.when(i != 0)
    def _():
        sum_ref[...] += s
        sq_ref[...] += sq


def _bn_relu(q, sum_ref, sq_ref, g_ref, bt_ref, p):
    sl = slice(p * 128, (p + 1) * 128)
    m = sum_ref[p:p + 1, :] * (1.0 / N_NODES)
    v = sq_ref[p:p + 1, :] * (1.0 / N_NODES) - m * m
    hn = (q - m) * lax.rsqrt(v + EPS) * g_ref[0, sl] + bt_ref[0, sl]
    return jnp.maximum(hn, 0.0)


def _tc_b_body(agg_ref, hs_ref, degp_ref, sum_ref, sq_ref,
               g_ref, bt_ref, w_ref, hsn_ref):
    dinv = _dinv_block(degp_ref[...])
    h0 = _bn_relu(_q_block(agg_ref, hs_ref, dinv, 0),
                  sum_ref, sq_ref, g_ref, bt_ref, 0)
    h1 = _bn_relu(_q_block(agg_ref, hs_ref, dinv, 1),
                  sum_ref, sq_ref, g_ref, bt_ref, 1)
    hs = (_dot(h0, w_ref[:128, :]) + _dot(h1, w_ref[128:, :])) * dinv
    hsn_ref[0] = hs[:, :128]
    hsn_ref[1] = hs[:, 128:]


def _tc_c_body(agg_ref, hs_ref, degp_ref, sum_ref, sq_ref,
               g_ref, bt_ref, batch_ref, ps0_ref, ps1_ref, cnt_ref):
    i = pl.program_id(0)
    dinv = _dinv_block(degp_ref[...])
    h0 = _bn_relu(_q_block(agg_ref, hs_ref, dinv, 0),
                  sum_ref, sq_ref, g_ref, bt_ref, 0)
    h1 = _bn_relu(_q_block(agg_ref, hs_ref, dinv, 1),
                  sum_ref, sq_ref, g_ref, bt_ref, 1)
    ids = lax.broadcasted_iota(jnp.int32, (N_GRAPHS, BLK), 0)
    pt = jnp.where(ids == batch_ref[0], 1.0, 0.0)
    s0 = _dot(pt, h0)
    s1 = _dot(pt, h1)
    cnt = jnp.sum(pt, axis=1, keepdims=True)

    @pl.when(i == 0)
    def _():
        ps0_ref[...] = s0
        ps1_ref[...] = s1
        cnt_ref[...] = cnt

    @pl.when(i != 0)
    def _():
        ps0_ref[...] += s0
        ps1_ref[...] += s1
        cnt_ref[...] += cnt


def _tc_c2_body(ps0_ref, ps1_ref, cnt_ref, wc_ref, bc_ref, out_ref):
    cnt = jnp.maximum(cnt_ref[...], 1.0)
    p0 = ps0_ref[...] / cnt
    p1 = ps1_ref[...] / cnt
    out_ref[...] = _dot(p0, wc_ref[:128, :]) + _dot(p1, wc_ref[128:, :]) \
        + bc_ref[...]


def _row_spec(last):
    return pl.BlockSpec((BLK, last), lambda i: (i, 0))


_AGG_SPEC = pl.BlockSpec((2, BLK, 128), lambda i: (0, i, 0))
_DEGP_SPEC = pl.BlockSpec((2, BLK, 128), lambda i: (0, i, 0))


def _const_spec(shape):
    nd = len(shape)
    return pl.BlockSpec(shape, lambda i: (0,) * nd)


_HS_SPEC = pl.BlockSpec((2, BLK, 128), lambda i: (0, i, 0))
_HS_SHAPE = jax.ShapeDtypeStruct((2, N_NODES, 128), jnp.float32)


def _tc_a(x, w1, degp):
    return pl.pallas_call(
        _tc_a_body,
        grid=(GRID,),
        in_specs=[_row_spec(128), _const_spec((128, 256)), _DEGP_SPEC],
        out_specs=_HS_SPEC,
        out_shape=_HS_SHAPE,
    )(x, w1, degp)


def _tc_stats(agg, hs, degp):
    return pl.pallas_call(
        _stats_body,
        grid=(GRID,),
        in_specs=[_AGG_SPEC, _HS_SPEC, _DEGP_SPEC],
        out_specs=(_const_spec((2, 128)), _const_spec((2, 128))),
        out_shape=(jax.ShapeDtypeStruct((2, 128), jnp.float32),
                   jax.ShapeDtypeStruct((2, 128), jnp.float32)),
    )(agg, hs, degp)


def _tc_b(agg, hs, degp, g, bt, w):
    s, sq = _tc_stats(agg, hs, degp)
    return pl.pallas_call(
        _tc_b_body,
        grid=(GRID,),
        in_specs=[_AGG_SPEC, _HS_SPEC, _DEGP_SPEC,
                  _const_spec((2, 128)), _const_spec((2, 128)),
                  _const_spec((1, 256)), _const_spec((1, 256)),
                  _const_spec((256, 256))],
        out_specs=_HS_SPEC,
        out_shape=_HS_SHAPE,
    )(agg, hs, degp, s, sq, g, bt, w)


def _tc_c(agg, hs, degp, g, bt, batch2d, wc, bc):
    s, sq = _tc_stats(agg, hs, degp)
    ps0, ps1, cnt = pl.pallas_call(
        _tc_c_body,
        grid=(GRID,),
        in_specs=[_AGG_SPEC, _HS_SPEC, _DEGP_SPEC,
                  _const_spec((2, 128)), _const_spec((2, 128)),
                  _const_spec((1, 256)), _const_spec((1, 256)),
                  pl.BlockSpec((1, 1, BLK), lambda i: (i, 0, 0))],
        out_specs=(_const_spec((N_GRAPHS, 128)), _const_spec((N_GRAPHS, 128)),
                   _const_spec((N_GRAPHS, 1))),
        out_shape=(jax.ShapeDtypeStruct((N_GRAPHS, 128), jnp.float32),
                   jax.ShapeDtypeStruct((N_GRAPHS, 128), jnp.float32),
                   jax.ShapeDtypeStruct((N_GRAPHS, 1), jnp.float32)),
    )(agg, hs, degp, s, sq, g, bt, batch2d)
    return pl.pallas_call(
        _tc_c2_body,
        out_shape=jax.ShapeDtypeStruct((N_GRAPHS, 2), jnp.float32),
    )(ps0, ps1, cnt, wc, bc)


@functools.cache
def _sc_kernels():
    mesh = plsc.VectorSubcoreMesh(core_axis_name="c", subcore_axis_name="s",
                                  num_cores=2, num_subcores=16)

    @functools.partial(
        pl.kernel,
        out_type=jax.ShapeDtypeStruct((2, NROWS, 128), jnp.float32),
        mesh=mesh,
        scratch_types=[
            pltpu.VMEM((DEG_K,), jnp.int32),
            pltpu.VMEM((DEG_K, 128), jnp.float32),
            pltpu.VMEM_SHARED((NROWS, 128), jnp.float32),
        ],
    )
    def _deg_kernel(dstp_hbm, ones_hbm, zeros_hbm, out_hbm, didx, ones_v, acc):
        c = lax.axis_index("c")
        s = lax.axis_index("s")
        pltpu.sync_copy(zeros_hbm,
                        acc.at[pl.ds(s * ROWS_PER_TILE, ROWS_PER_TILE)])
        pltpu.sync_copy(ones_hbm, ones_v)
        plsc.subcore_barrier()
        base = (s * 2 + c) * DEG_PER_TILE

        def body(i, carry):
            off = base + i * DEG_K
            pltpu.sync_copy(dstp_hbm.at[pl.ds(off, DEG_K)], didx)
            pltpu.sync_copy(ones_v, acc.at[didx], add=True)
            return carry

        lax.fori_loop(0, DEG_CHUNKS, body, 0)
        plsc.subcore_barrier()
        sl = pl.ds(s * ROWS_PER_TILE, ROWS_PER_TILE)
        pltpu.sync_copy(acc.at[sl], out_hbm.at[c, sl])

    @functools.partial(
        pl.kernel,
        out_type=jax.ShapeDtypeStruct((2, NROWS, 128), jnp.float32),
        mesh=mesh,
        scratch_types=[
            pltpu.VMEM((AGG_K,), jnp.int32),
            pltpu.VMEM((AGG_K,), jnp.int32),
            pltpu.VMEM((AGG_K, 128), jnp.float32),
            pltpu.VMEM_SHARED((NROWS, 128), jnp.float32),
        ],
    )
    def _agg_kernel(hs_hbm, srcp_hbm, dstp_hbm, zeros_hbm, out_hbm,
                    sidx, didx, rows, acc):
        c = lax.axis_index("c")
        s = lax.axis_index("s")
        for i in range(ROWS_PER_TILE // 128):
            pltpu.sync_copy(zeros_hbm,
                            acc.at[pl.ds(s * ROWS_PER_TILE + i * 128, 128)])
        plsc.subcore_barrier()
        base = s * (AGG_CHUNKS * AGG_K)

        def body(i, carry):
            off = base + i * AGG_K
            pltpu.sync_copy(srcp_hbm.at[pl.ds(off, AGG_K)], sidx)
            pltpu.sync_copy(dstp_hbm.at[pl.ds(off, AGG_K)], didx)

            pltpu.sync_copy(hs_hbm.at[c].at[sidx], rows)

            pltpu.sync_copy(rows, acc.at[didx], add=True)
            return carry

        lax.fori_loop(0, AGG_CHUNKS, body, 0)
        plsc.subcore_barrier()
        sl = pl.ds(s * ROWS_PER_TILE, ROWS_PER_TILE)
        pltpu.sync_copy(acc.at[sl], out_hbm.at[c, sl])

    return _deg_kernel, _agg_kernel


def kernel(x, edge_index, batch, W1, b1, g1, bt1, W2, b2, g2, bt2,
           W3, b3, g3, bt3, Wc, bc):
    src = edge_index[0]
    dst = edge_index[1]
    pad = E_PAD - N_EDGES
    srcp = jnp.concatenate([src, jnp.zeros((pad,), src.dtype)])
    dstp = jnp.concatenate([dst, jnp.full((pad,), TRASH, dst.dtype)])
    zeros128 = jnp.zeros((128, 128), jnp.float32)
    zeros_deg = jnp.zeros((ROWS_PER_TILE, 128), jnp.float32)
    ones_deg = jnp.ones((DEG_K, 128), jnp.float32)

    del b1, b2, b3

    _deg_kernel, _agg_kernel = _sc_kernels()
    degp = _deg_kernel(dstp, ones_deg, zeros_deg)
    hs = _tc_a(x, W1, degp)

    g1r, bt1r = g1.reshape(1, -1), bt1.reshape(1, -1)
    g2r, bt2r = g2.reshape(1, -1), bt2.reshape(1, -1)
    g3r, bt3r = g3.reshape(1, -1), bt3.reshape(1, -1)

    agg = _agg_kernel(hs, srcp, dstp, zeros128)
    hs = _tc_b(agg, hs, degp, g1r, bt1r, W2)
    agg = _agg_kernel(hs, srcp, dstp, zeros128)
    hs = _tc_b(agg, hs, degp, g2r, bt2r, W3)
    agg = _agg_kernel(hs, srcp, dstp, zeros128)
    return _tc_c(agg, hs, degp, g3r, bt3r,
                 batch.reshape(GRID, 1, BLK), Wc, bc.reshape(1, -1))
